# baseline (speedup 1.0000x reference)
import functools

import jax
import jax.numpy as jnp
from jax import lax
from jax.experimental import pallas as pl
from jax.experimental.pallas import tpu as pltpu
from jax.experimental.pallas import tpu_sc as plsc

_CHUNK = 128


@functools.lru_cache(maxsize=None)
def _make_sc_gather(n_chunks, d, v):
    info = plsc.get_sparse_core_info()
    nc, ns = info.num_cores, info.num_subcores
    nw = nc * ns
    per_w = n_chunks // nw

    mesh = plsc.VectorSubcoreMesh(core_axis_name="c", subcore_axis_name="s")

    @functools.partial(
        pl.kernel,
        mesh=mesh,
        out_type=jax.ShapeDtypeStruct((n_chunks, _CHUNK, d), jnp.float32),
        scratch_types=[
            pltpu.VMEM((per_w, _CHUNK), jnp.int32),
            pltpu.VMEM((per_w, _CHUNK, d), jnp.float32),
            pltpu.SemaphoreType.DMA,
        ],
        compiler_params=pltpu.CompilerParams(use_tc_tiling_on_sc=False),
    )
    def gather_kernel(tok_hbm, idx_hbm, out_hbm, idx_v, rows_v, sem):
        wid = lax.axis_index("s") * nc + lax.axis_index("c")
        base = wid * per_w
        pltpu.sync_copy(idx_hbm.at[pl.ds(base, per_w)], idx_v)
        copies = [
            pltpu.async_copy(tok_hbm.at[idx_v.at[k]], rows_v.at[k], sem)
            for k in range(per_w)
        ]
        for c in copies:
            c.wait()
        pltpu.sync_copy(rows_v, out_hbm.at[pl.ds(base, per_w)])

    return gather_kernel


def _matmul_body(x_ref, w_ref, b_ref, pos_ref, o_ref, *, bt, t, vt):
    w = w_ref[:]
    acc = jnp.dot(x_ref[:], w, preferred_element_type=jnp.float32)
    p = jnp.dot(pos_ref[:], w, preferred_element_type=jnp.float32) + b_ref[:]
    pt = jnp.broadcast_to(p[None, :, :], (bt // t, t, vt)).reshape(bt, vt)
    o_ref[:] = acc + pt


def _lm_head(x, W, b2, pos_table, *, bt, vt):
    bf, d = x.shape
    t = pos_table.shape[0]
    v = W.shape[1]
    grid = (bf // bt, pl.cdiv(v, vt))
    return pl.pallas_call(
        functools.partial(_matmul_body, bt=bt, t=t, vt=vt),
        grid=grid,
        in_specs=[
            pl.BlockSpec((bt, d), lambda i, j: (i, 0)),
            pl.BlockSpec((d, vt), lambda i, j: (0, j)),
            pl.BlockSpec((1, vt), lambda i, j: (0, j)),
            pl.BlockSpec((t, d), lambda i, j: (0, 0)),
        ],
        out_specs=pl.BlockSpec((bt, vt), lambda i, j: (i, j)),
        out_shape=jax.ShapeDtypeStruct((bf, v), jnp.float32),
        compiler_params=pltpu.CompilerParams(
            dimension_semantics=("parallel", "parallel"),
        ),
    )(x, W, b2, pos_table)


def kernel(idx, tok_table, pos_table, W, b):
    B, T = idx.shape
    V, D = tok_table.shape
    bf = B * T
    n_chunks = bf // _CHUNK
    idx2d = idx.reshape(n_chunks, _CHUNK).astype(jnp.int32)
    x = _make_sc_gather(n_chunks, D, V)(tok_table, idx2d)
    x2 = x.reshape(bf, D)
    out2 = _lm_head(x2, W, b.reshape(1, V), pos_table, bt=1024, vt=2048)
    return out2.reshape(B, T, V)
